# baseline (speedup 1.0000x reference)
import functools

import jax
import jax.numpy as jnp
from jax import lax
from jax.experimental import pallas as pl
from jax.experimental.pallas import tpu as pltpu
from jax.experimental.pallas import tpu_sc as plsc

N_NODES = 10000
D = 128
N_EDGES = 320000

NC, NS = 2, 16
NW = NC * NS
CHUNK = 128
E_PAD = 327680
NCHUNK = E_PAD // (NW * CHUNK)
EPW = NCHUNK * CHUNK
TOT_CHUNKS = E_PAD // CHUNK
K0, K1 = 96, 64
IBLK = 16
PAD_DST = 10016
ACC_ROWS = 10240
ZROWS = ACC_ROWS // NS
HR, HC = ACC_ROWS // 16, 16


def _deg_sc_body(dst_hbm, out_hbm, idx_v, hist_v):
    c = lax.axis_index("c")
    s = lax.axis_index("s")
    w = c * NS + s

    @pl.loop(0, HR)
    def _(i):
        hist_v[i] = jnp.zeros((16,), jnp.float32)

    pltpu.sync_copy(dst_hbm.at[w], idx_v)
    ones = jnp.full((16,), 1.0, jnp.float32)

    @pl.loop(0, EPW, step=16)
    def _(i):
        vec = idx_v[pl.ds(i, 16)]
        row = lax.shift_right_logical(vec, 4)
        col = lax.bitwise_and(vec, 15)
        plsc.addupdate_scatter(hist_v, [row, col], ones)

    pltpu.sync_copy(hist_v, out_hbm.at[c, s])


PDEPTH = 2


def _segsum_sc_body(g_hbm, src_hbm, dst_hbm, out_hbm,
                    sv, dv, rw0, rw1, acc_sh,
                    si, sg0, sg1, ss0, ss1):
    c = lax.axis_index("c")
    s = lax.axis_index("s")
    rws = (rw0, rw1)
    sgs = (sg0, sg1)
    sss = (ss0, ss1)

    @pl.loop(0, CHUNK)
    def _(i):
        @pl.loop(0, D, step=16)
        def _(k):
            rw0[i, pl.ds(k, 16)] = jnp.zeros((16,), jnp.float32)

    @pl.loop(0, ZROWS // CHUNK)
    def _(k):
        pltpu.sync_copy(rw0, acc_sh.at[pl.ds(s * ZROWS + k * CHUNK, CHUNK)])

    plsc.subcore_barrier()

    def _run(base, nch):
        @pl.loop(0, nch, step=IBLK)
        def _(j):
            jj = base + j
            pltpu.async_copy(src_hbm.at[pl.ds(jj, IBLK)], sv, si).wait()
            pltpu.async_copy(dst_hbm.at[pl.ds(jj, IBLK)], dv, si).wait()
            scat = [None] * IBLK
            for n in range(IBLK):
                p = n % 2
                if n >= 2:
                    scat[n - 2].wait()
                pltpu.async_copy(g_hbm.at[sv.at[n]], rws[p], sgs[p]).wait()
                scat[n] = pltpu.async_copy(rws[p], acc_sh.at[dv.at[n]],
                                           sss[p], add=True)
            scat[IBLK - 2].wait()
            scat[IBLK - 1].wait()

    @pl.when(c == 0)
    def _():
        _run(s * K0, K0)

    @pl.when(c == 1)
    def _():
        _run(NS * K0 + s * K1, K1)

    plsc.subcore_barrier()
    pltpu.sync_copy(acc_sh.at[pl.ds(s * ZROWS, ZROWS)],
                    out_hbm.at[c, pl.ds(s * ZROWS, ZROWS)])


def _tc_degsum_body(degp_ref, dis_ref):
    deg = jnp.sum(degp_ref[...], axis=0) + 1.0
    dis_ref[...] = lax.rsqrt(deg)


def _tc_g1_body(x_ref, w_ref, dis_ref, g_ref):
    h = jnp.dot(x_ref[...], w_ref[...], preferred_element_type=jnp.float32)
    g_ref[...] = h * dis_ref[...]


def _tc_mid_body(accp_ref, g_ref, dis_ref, b1_ref, gnw_ref, gnb_ref,
                 gnms_ref, w2_ref, g2_ref):
    dis = dis_ref[...]
    c1 = ((accp_ref[0, :N_NODES] + accp_ref[1, :N_NODES] + g_ref[...])
          * dis + b1_ref[...])
    mean = jnp.mean(c1, axis=0, keepdims=True)
    o = c1 - mean * gnms_ref[...]
    var = jnp.mean(o * o, axis=0, keepdims=True)
    y = gnw_ref[...] * o / jnp.sqrt(var + 1e-5) + gnb_ref[...]
    y = jnp.maximum(y, 0.0)
    h2 = jnp.dot(y, w2_ref[...], preferred_element_type=jnp.float32)
    g2_ref[...] = h2 * dis


def _tc_out_body(accp_ref, g_ref, dis_ref, b2_ref, out_ref):
    out_ref[...] = ((accp_ref[0, :N_NODES] + accp_ref[1, :N_NODES]
                     + g_ref[...]) * dis_ref[...] + b2_ref[...])


_f32 = jnp.float32

_tc_degsum = pl.pallas_call(
    _tc_degsum_body,
    out_shape=jax.ShapeDtypeStruct((HR, HC), _f32),
)

_tc_g1 = pl.pallas_call(
    _tc_g1_body,
    out_shape=jax.ShapeDtypeStruct((N_NODES, D), _f32),
)

_tc_mid = pl.pallas_call(
    _tc_mid_body,
    out_shape=jax.ShapeDtypeStruct((N_NODES, D), _f32),
)

_tc_out = pl.pallas_call(
    _tc_out_body,
    out_shape=jax.ShapeDtypeStruct((N_NODES, D), _f32),
)


@functools.cache
def _sc_kernels():
    mesh = plsc.VectorSubcoreMesh(core_axis_name="c", subcore_axis_name="s")
    deg = pl.kernel(
        _deg_sc_body,
        mesh=mesh,
        out_type=jax.ShapeDtypeStruct((NC, NS, HR, HC), jnp.float32),
        scratch_types=[
            pltpu.VMEM((EPW,), jnp.int32),
            pltpu.VMEM((HR, HC), jnp.float32),
        ],
        compiler_params=pltpu.CompilerParams(needs_layout_passes=False),
    )
    segsum = pl.kernel(
        _segsum_sc_body,
        mesh=mesh,
        out_type=jax.ShapeDtypeStruct((NC, ACC_ROWS, D), jnp.float32),
        scratch_types=(
            [pltpu.VMEM((IBLK, CHUNK), jnp.int32) for _ in range(2)]
            + [pltpu.VMEM((CHUNK, D), jnp.float32) for _ in range(PDEPTH)]
            + [pltpu.VMEM_SHARED((ACC_ROWS, D), jnp.float32)]
            + [pltpu.SemaphoreType.DMA for _ in range(1 + 2 * PDEPTH)]
        ),
    )
    return deg, segsum


def kernel(x, edge_index, W1, b1, gn_w, gn_b, gn_ms, W2, b2):
    _deg_sc, _segsum_sc = _sc_kernels()
    src = edge_index[0].astype(jnp.int32)
    dst = edge_index[1].astype(jnp.int32)
    pad = E_PAD - N_EDGES
    src_mat = jnp.concatenate(
        [src, jnp.zeros((pad,), jnp.int32)]).reshape(TOT_CHUNKS, CHUNK)
    dst_mat = jnp.concatenate(
        [dst, jnp.full((pad,), PAD_DST, jnp.int32)]).reshape(TOT_CHUNKS, CHUNK)

    degp = _deg_sc(dst_mat.reshape(NW, EPW))
    dis_blk = _tc_degsum(degp.reshape(NW, HR, HC))
    dis_col = dis_blk.reshape(ACC_ROWS, 1)[:N_NODES]

    b1r = b1.reshape(1, D)
    b2r = b2.reshape(1, D)
    gnwr = gn_w.reshape(1, D)
    gnbr = gn_b.reshape(1, D)
    gnmsr = gn_ms.reshape(1, D)

    g1 = _tc_g1(x, W1, dis_col)
    acc1 = _segsum_sc(g1, src_mat, dst_mat)
    g2 = _tc_mid(acc1, g1, dis_col, b1r, gnwr, gnbr, gnmsr, W2)
    acc2 = _segsum_sc(g2, src_mat, dst_mat)
    return _tc_out(acc2, g2, dis_col, b2r)


# speedup vs baseline: 1.0581x; 1.0581x over previous
import functools

import jax
import jax.numpy as jnp
from jax import lax
from jax.experimental import pallas as pl
from jax.experimental.pallas import tpu as pltpu
from jax.experimental.pallas import tpu_sc as plsc

N_NODES = 10000
D = 128
N_EDGES = 320000

NC, NS = 2, 16
NW = NC * NS
CHUNK = 128
E_PAD = 327680
NCHUNK = E_PAD // (NW * CHUNK)
EPW = NCHUNK * CHUNK
TOT_CHUNKS = E_PAD // CHUNK
K0, K1 = 112, 48
IBLK = 16
PAD_DST = 10016
ACC_ROWS = 10240
ZROWS = ACC_ROWS // NS
HR, HC = ACC_ROWS // 16, 16


def _deg_sc_body(dst_hbm, out_hbm, idx_v, hist_v):
    c = lax.axis_index("c")
    s = lax.axis_index("s")
    w = c * NS + s

    @pl.loop(0, HR)
    def _(i):
        hist_v[i] = jnp.zeros((16,), jnp.float32)

    pltpu.sync_copy(dst_hbm.at[w], idx_v)
    ones = jnp.full((16,), 1.0, jnp.float32)

    @pl.loop(0, EPW, step=16)
    def _(i):
        vec = idx_v[pl.ds(i, 16)]
        row = lax.shift_right_logical(vec, 4)
        col = lax.bitwise_and(vec, 15)
        plsc.addupdate_scatter(hist_v, [row, col], ones)

    pltpu.sync_copy(hist_v, out_hbm.at[c, s])


PDEPTH = 2


def _segsum_sc_body(g_hbm, src_hbm, dst_hbm, out_hbm,
                    sv, dv, rw0, rw1, acc_sh,
                    si, sg0, sg1, ss0, ss1):
    c = lax.axis_index("c")
    s = lax.axis_index("s")
    rws = (rw0, rw1)
    sgs = (sg0, sg1)
    sss = (ss0, ss1)

    @pl.loop(0, CHUNK)
    def _(i):
        @pl.loop(0, D, step=16)
        def _(k):
            rw0[i, pl.ds(k, 16)] = jnp.zeros((16,), jnp.float32)

    @pl.loop(0, ZROWS // CHUNK)
    def _(k):
        pltpu.sync_copy(rw0, acc_sh.at[pl.ds(s * ZROWS + k * CHUNK, CHUNK)])

    plsc.subcore_barrier()

    def _run(base, nch):
        @pl.loop(0, nch, step=IBLK)
        def _(j):
            jj = base + j
            pltpu.async_copy(src_hbm.at[pl.ds(jj, IBLK)], sv, si).wait()
            pltpu.async_copy(dst_hbm.at[pl.ds(jj, IBLK)], dv, si).wait()
            scat = [None] * IBLK
            for n in range(IBLK):
                p = n % 2
                if n >= 2:
                    scat[n - 2].wait()
                pltpu.async_copy(g_hbm.at[sv.at[n]], rws[p], sgs[p]).wait()
                scat[n] = pltpu.async_copy(rws[p], acc_sh.at[dv.at[n]],
                                           sss[p], add=True)
            scat[IBLK - 2].wait()
            scat[IBLK - 1].wait()

    @pl.when(c == 0)
    def _():
        _run(s * K0, K0)

    @pl.when(c == 1)
    def _():
        _run(NS * K0 + s * K1, K1)

    plsc.subcore_barrier()
    pltpu.sync_copy(acc_sh.at[pl.ds(s * ZROWS, ZROWS)],
                    out_hbm.at[c, pl.ds(s * ZROWS, ZROWS)])


def _tc_degsum_body(degp_ref, dis_ref):
    deg = jnp.sum(degp_ref[...], axis=0) + 1.0
    dis_ref[...] = lax.rsqrt(deg)


def _tc_g1_body(x_ref, w_ref, dis_ref, g_ref):
    h = jnp.dot(x_ref[...], w_ref[...], preferred_element_type=jnp.float32)
    g_ref[...] = h * dis_ref[...]


def _tc_mid_body(accp_ref, g_ref, dis_ref, b1_ref, gnw_ref, gnb_ref,
                 gnms_ref, w2_ref, g2_ref):
    dis = dis_ref[...]
    c1 = ((accp_ref[0, :N_NODES] + accp_ref[1, :N_NODES] + g_ref[...])
          * dis + b1_ref[...])
    mean = jnp.mean(c1, axis=0, keepdims=True)
    o = c1 - mean * gnms_ref[...]
    var = jnp.mean(o * o, axis=0, keepdims=True)
    y = gnw_ref[...] * o / jnp.sqrt(var + 1e-5) + gnb_ref[...]
    y = jnp.maximum(y, 0.0)
    h2 = jnp.dot(y, w2_ref[...], preferred_element_type=jnp.float32)
    g2_ref[...] = h2 * dis


def _tc_out_body(accp_ref, g_ref, dis_ref, b2_ref, out_ref):
    out_ref[...] = ((accp_ref[0, :N_NODES] + accp_ref[1, :N_NODES]
                     + g_ref[...]) * dis_ref[...] + b2_ref[...])


_f32 = jnp.float32

_tc_degsum = pl.pallas_call(
    _tc_degsum_body,
    out_shape=jax.ShapeDtypeStruct((HR, HC), _f32),
)

_tc_g1 = pl.pallas_call(
    _tc_g1_body,
    out_shape=jax.ShapeDtypeStruct((N_NODES, D), _f32),
)

_tc_mid = pl.pallas_call(
    _tc_mid_body,
    out_shape=jax.ShapeDtypeStruct((N_NODES, D), _f32),
)

_tc_out = pl.pallas_call(
    _tc_out_body,
    out_shape=jax.ShapeDtypeStruct((N_NODES, D), _f32),
)


@functools.cache
def _sc_kernels():
    mesh = plsc.VectorSubcoreMesh(core_axis_name="c", subcore_axis_name="s")
    deg = pl.kernel(
        _deg_sc_body,
        mesh=mesh,
        out_type=jax.ShapeDtypeStruct((NC, NS, HR, HC), jnp.float32),
        scratch_types=[
            pltpu.VMEM((EPW,), jnp.int32),
            pltpu.VMEM((HR, HC), jnp.float32),
        ],
        compiler_params=pltpu.CompilerParams(needs_layout_passes=False),
    )
    segsum = pl.kernel(
        _segsum_sc_body,
        mesh=mesh,
        out_type=jax.ShapeDtypeStruct((NC, ACC_ROWS, D), jnp.float32),
        scratch_types=(
            [pltpu.VMEM((IBLK, CHUNK), jnp.int32) for _ in range(2)]
            + [pltpu.VMEM((CHUNK, D), jnp.float32) for _ in range(PDEPTH)]
            + [pltpu.VMEM_SHARED((ACC_ROWS, D), jnp.float32)]
            + [pltpu.SemaphoreType.DMA for _ in range(1 + 2 * PDEPTH)]
        ),
    )
    return deg, segsum


def kernel(x, edge_index, W1, b1, gn_w, gn_b, gn_ms, W2, b2):
    _deg_sc, _segsum_sc = _sc_kernels()
    src = edge_index[0].astype(jnp.int32)
    dst = edge_index[1].astype(jnp.int32)
    pad = E_PAD - N_EDGES
    src_mat = jnp.concatenate(
        [src, jnp.zeros((pad,), jnp.int32)]).reshape(TOT_CHUNKS, CHUNK)
    dst_mat = jnp.concatenate(
        [dst, jnp.full((pad,), PAD_DST, jnp.int32)]).reshape(TOT_CHUNKS, CHUNK)

    degp = _deg_sc(dst_mat.reshape(NW, EPW))
    dis_blk = _tc_degsum(degp.reshape(NW, HR, HC))
    dis_col = dis_blk.reshape(ACC_ROWS, 1)[:N_NODES]

    b1r = b1.reshape(1, D)
    b2r = b2.reshape(1, D)
    gnwr = gn_w.reshape(1, D)
    gnbr = gn_b.reshape(1, D)
    gnmsr = gn_ms.reshape(1, D)

    g1 = _tc_g1(x, W1, dis_col)
    acc1 = _segsum_sc(g1, src_mat, dst_mat)
    g2 = _tc_mid(acc1, g1, dis_col, b1r, gnwr, gnbr, gnmsr, W2)
    acc2 = _segsum_sc(g2, src_mat, dst_mat)
    return _tc_out(acc2, g2, dis_col, b2r)
